# Initial kernel scaffold; baseline (speedup 1.0000x reference)
import jax
import jax.numpy as jnp
from jax.experimental import pallas as pl


def kernel(batch_inputs, embedding_matrix, synonym_indices, Wk_f, Wr_f, b_f, Wk_b, Wr_b, b_b, Wp, bp, Ws, bs):
    raise NotImplementedError("write your pallas kernel here")


# speedup vs baseline: 1.1703x; 1.1703x over previous
import jax
import jax.numpy as jnp
from jax.experimental import pallas as pl
from jax.experimental.pallas import tpu as pltpu

SEQ = 50
EMB = 300
NSYN = 4
UNITS = 150
GP = 256


def _body(sent_ref, table_ref, syntab_ref,
          wkf_ref, wkb_ref, bkf_ref, bkb_ref, wrf_ref, wrb_ref,
          wp_ref, bp_ref, ws_ref, bs_ref, out_ref,
          emb_v, synidx_sm, syn_v, xf_ref, xb_ref, hf_ref, hb_ref,
          emb_sem, si_sem, syn_sem):
    f32 = jnp.float32

    emb_copies = []
    si_copies = []
    for i in range(SEQ):
        s = sent_ref[i]
        c = pltpu.make_async_copy(
            table_ref.at[pl.ds(s, 1)], emb_v.at[pl.ds(i, 1)], emb_sem)
        c.start()
        emb_copies.append(c)
        c2 = pltpu.make_async_copy(
            syntab_ref.at[pl.ds(s, 1)], synidx_sm.at[pl.ds(i, 1)], si_sem)
        c2.start()
        si_copies.append(c2)
    for c in si_copies:
        c.wait()

    syn_copies = []
    for s in range(SEQ):
        for w in range(NSYN):
            r = synidx_sm[s, w]
            c = pltpu.make_async_copy(
                table_ref.at[pl.ds(r, 1)], syn_v.at[w, pl.ds(s, 1)], syn_sem)
            c.start()
            syn_copies.append(c)
    for c in emb_copies:
        c.wait()

    emb = emb_v[...]
    xf_ref[...] = jnp.dot(emb, wkf_ref[...],
                          preferred_element_type=f32) + bkf_ref[...]
    xb_ref[...] = jnp.dot(emb, wkb_ref[...],
                          preferred_element_type=f32) + bkb_ref[...]

    hf = jnp.zeros((1, GP), f32)
    cf = jnp.zeros((1, GP), f32)
    hb = jnp.zeros((1, GP), f32)
    cb = jnp.zeros((1, GP), f32)
    for t in range(SEQ):
        tb = SEQ - 1 - t
        zf = jnp.dot(hf, wrf_ref[...], preferred_element_type=f32) \
            + xf_ref[t:t + 1, :]
        zb = jnp.dot(hb, wrb_ref[...], preferred_element_type=f32) \
            + xb_ref[tb:tb + 1, :]
        i_f = jax.nn.sigmoid(zf[:, 0 * GP:1 * GP])
        f_f = jax.nn.sigmoid(zf[:, 1 * GP:2 * GP])
        g_f = jnp.tanh(zf[:, 2 * GP:3 * GP])
        o_f = jax.nn.sigmoid(zf[:, 3 * GP:4 * GP])
        cf = f_f * cf + i_f * g_f
        hf = o_f * jnp.tanh(cf)
        hf_ref[t:t + 1, :] = hf
        i_b = jax.nn.sigmoid(zb[:, 0 * GP:1 * GP])
        f_b = jax.nn.sigmoid(zb[:, 1 * GP:2 * GP])
        g_b = jnp.tanh(zb[:, 2 * GP:3 * GP])
        o_b = jax.nn.sigmoid(zb[:, 3 * GP:4 * GP])
        cb = f_b * cb + i_b * g_b
        hb = o_b * jnp.tanh(cb)
        hb_ref[tb:tb + 1, :] = hb

    hidden = jnp.concatenate(
        [hf_ref[:, 0:UNITS], hb_ref[:, 0:UNITS]], axis=1)
    out = jnp.dot(hidden, wp_ref[...], preferred_element_type=f32) \
        + bp_ref[...]

    for c in syn_copies:
        c.wait()
    m = jnp.zeros((SEQ, EMB), f32)
    for w in range(NSYN):
        sw = syn_v[w]
        cw = jnp.exp(jnp.sum(sw * out, axis=1, keepdims=True))
        m = m + cw * sw
    hh = m + hidden
    c2 = jnp.exp(jnp.tanh(
        jnp.sum(hh * ws_ref[...], axis=1, keepdims=True) + bs_ref[...]))
    out_ref[...] = jnp.sum(c2 * hh, axis=0, keepdims=True)


def _pad_gates(w, rows_used, rows_pad):
    w = jnp.pad(w, ((0, rows_pad - rows_used), (0, 0)))
    w = w.reshape(rows_pad, 4, UNITS)
    w = jnp.pad(w, ((0, 0), (0, 0), (0, GP - UNITS)))
    return w.reshape(rows_pad, 4 * GP)


def kernel(batch_inputs, embedding_matrix, synonym_indices,
           Wk_f, Wr_f, b_f, Wk_b, Wr_b, b_b, Wp, bp, Ws, bs):
    sent = batch_inputs[0].astype(jnp.int32)

    wkf = _pad_gates(Wk_f, EMB, EMB)
    wkb = _pad_gates(Wk_b, EMB, EMB)
    wrf = _pad_gates(Wr_f, UNITS, GP)
    wrb = _pad_gates(Wr_b, UNITS, GP)
    bkf = _pad_gates(b_f.reshape(1, 4 * UNITS), 1, 1)
    bkb = _pad_gates(b_b.reshape(1, 4 * UNITS), 1, 1)

    res = pl.pallas_call(
        _body,
        out_shape=jax.ShapeDtypeStruct((1, EMB), jnp.float32),
        in_specs=[
            pl.BlockSpec(memory_space=pltpu.SMEM),
            pl.BlockSpec(memory_space=pl.ANY),
            pl.BlockSpec(memory_space=pl.ANY),
        ] + [pl.BlockSpec(memory_space=pltpu.VMEM)] * 10,
        scratch_shapes=[
            pltpu.VMEM((SEQ, EMB), jnp.float32),
            pltpu.SMEM((SEQ, NSYN), jnp.int32),
            pltpu.VMEM((NSYN, SEQ, EMB), jnp.float32),
            pltpu.VMEM((SEQ, 4 * GP), jnp.float32),
            pltpu.VMEM((SEQ, 4 * GP), jnp.float32),
            pltpu.VMEM((SEQ, GP), jnp.float32),
            pltpu.VMEM((SEQ, GP), jnp.float32),
            pltpu.SemaphoreType.DMA,
            pltpu.SemaphoreType.DMA,
            pltpu.SemaphoreType.DMA,
        ],
    )(sent, embedding_matrix, synonym_indices.astype(jnp.int32),
      wkf, wkb, bkf, bkb, wrf, wrb,
      Wp, bp.reshape(1, EMB), Ws.reshape(1, EMB), bs.reshape(1, 1))
    return res.reshape(EMB)
